# baseline (device time: 403377 ns/iter reference)
import jax
import jax.numpy as jnp
from jax import lax
from jax.experimental import pallas as pl
from jax.experimental.pallas import tpu as pltpu

N_DEV = 32


def kernel(x, w_mat):
    m, k_loc = x.shape
    _, n = w_mat.shape
    cm = m // N_DEV

    def body(x_ref, w_ref, out_ref, acc_ref, recv_ref,
             send_sem, recv_sem, credit_sem):
        my = lax.axis_index("i")
        left = lax.rem(my + N_DEV - 1, N_DEV)
        right = lax.rem(my + 1, N_DEV)

        barrier = pltpu.get_barrier_semaphore()
        for nbr in (left, right):
            pl.semaphore_signal(barrier, inc=1, device_id=(nbr,),
                                device_id_type=pl.DeviceIdType.MESH)
        pl.semaphore_wait(barrier, 2)

        def contrib(c):
            xa = x_ref[pl.ds(c * cm, cm), :]
            return jnp.dot(xa, w_ref[...], preferred_element_type=jnp.float32)

        acc_ref[...] = contrib(left)

        for s in range(N_DEV - 1):
            rdma = pltpu.make_async_remote_copy(
                src_ref=acc_ref, dst_ref=recv_ref,
                send_sem=send_sem, recv_sem=recv_sem,
                device_id=(right,), device_id_type=pl.DeviceIdType.MESH,
            )
            rdma.start()
            rdma.wait()

            if s < N_DEV - 2:
                c = lax.rem(my + 2 * N_DEV - s - 2, N_DEV)
                acc_ref[...] = recv_ref[...] + contrib(c)
                pl.semaphore_signal(credit_sem, inc=1, device_id=(left,),
                                    device_id_type=pl.DeviceIdType.MESH)
                pl.semaphore_wait(credit_sem, 1)
            else:
                out_ref[...] = jnp.maximum(recv_ref[...] + contrib(my), 0.0)

    return pl.pallas_call(
        body,
        out_shape=jax.ShapeDtypeStruct((cm, n), jnp.float32),
        in_specs=[
            pl.BlockSpec(memory_space=pltpu.VMEM),
            pl.BlockSpec(memory_space=pltpu.VMEM),
        ],
        out_specs=pl.BlockSpec(memory_space=pltpu.VMEM),
        scratch_shapes=[
            pltpu.VMEM((cm, n), jnp.float32),
            pltpu.VMEM((cm, n), jnp.float32),
            pltpu.SemaphoreType.DMA,
            pltpu.SemaphoreType.DMA,
            pltpu.SemaphoreType.REGULAR,
        ],
        compiler_params=pltpu.CompilerParams(collective_id=0),
    )(x, w_mat)


# device time: 228897 ns/iter; 1.7623x vs baseline; 1.7623x over previous
import jax
import jax.numpy as jnp
from jax import lax
from jax.experimental import pallas as pl
from jax.experimental.pallas import tpu as pltpu

N_DEV = 32
K_SUB = 2


def kernel(x, w_mat):
    m, k_loc = x.shape
    _, n = w_mat.shape
    cm = m // N_DEV
    half = n // 2
    subw = half // K_SUB

    def body(x_ref, w_ref, out_ref, acc_ref, recv_ref,
             send_sem, recv_sem, credit_sem):
        my = lax.axis_index("i")
        left = lax.rem(my + N_DEV - 1, N_DEV)
        right = lax.rem(my + 1, N_DEV)
        nbr_send = (right, left)
        nbr_recv = (left, right)

        barrier = pltpu.get_barrier_semaphore()
        for nbr in (left, right):
            pl.semaphore_signal(barrier, inc=1, device_id=(nbr,),
                                device_id_type=pl.DeviceIdType.MESH)
        pl.semaphore_wait(barrier, 2)

        def col0(d, j):
            return d * half + j * subw

        def contrib(c, d, j):
            xa = x_ref[pl.ds(c * cm, cm), :]
            wc = w_ref[:, col0(d, j):col0(d, j) + subw]
            return jnp.dot(xa, wc, preferred_element_type=jnp.float32)

        def send_chunk(d, s):
            if d == 0:
                return lax.rem(my + 2 * N_DEV - s - 1, N_DEV)
            return lax.rem(my + s + 1, N_DEV)

        chains = [(d, j) for d in range(2) for j in range(K_SUB)]

        for d, j in chains:
            acc_ref[d, j, 0, :, :] = contrib(send_chunk(d, 0), d, j)

        rdmas = {}
        for s in range(N_DEV - 1):
            slot = s % 2
            for d, j in chains:
                if s >= 2:
                    pl.semaphore_wait(credit_sem.at[d, j], 1)
                r = pltpu.make_async_remote_copy(
                    src_ref=acc_ref.at[d, j, slot],
                    dst_ref=recv_ref.at[d, j, slot],
                    send_sem=send_sem.at[d, j, slot],
                    recv_sem=recv_sem.at[d, j, slot],
                    device_id=(nbr_send[d],),
                    device_id_type=pl.DeviceIdType.MESH,
                )
                r.start()
                rdmas[(d, j, s)] = r

            if s < N_DEV - 2:
                tmps = {(d, j): contrib(send_chunk(d, s + 1), d, j)
                        for d, j in chains}
            else:
                tmps = {(d, j): contrib(my, d, j) for d, j in chains}

            for d, j in chains:
                rdmas[(d, j, s)].wait_recv()
                if s < N_DEV - 2:
                    if s >= 1:
                        rdmas[(d, j, s - 1)].wait_send()
                    acc_ref[d, j, (s + 1) % 2, :, :] = (
                        recv_ref[d, j, slot, :, :] + tmps[(d, j)]
                    )
                    if s <= N_DEV - 4:
                        pl.semaphore_signal(
                            credit_sem.at[d, j], inc=1,
                            device_id=(nbr_recv[d],),
                            device_id_type=pl.DeviceIdType.MESH,
                        )
                else:
                    out_ref[:, col0(d, j):col0(d, j) + subw] = jnp.maximum(
                        recv_ref[d, j, slot, :, :] + tmps[(d, j)], 0.0
                    )

        for d, j in chains:
            rdmas[(d, j, N_DEV - 3)].wait_send()
            rdmas[(d, j, N_DEV - 2)].wait_send()

    return pl.pallas_call(
        body,
        out_shape=jax.ShapeDtypeStruct((cm, n), jnp.float32),
        in_specs=[
            pl.BlockSpec(memory_space=pltpu.VMEM),
            pl.BlockSpec(memory_space=pltpu.VMEM),
        ],
        out_specs=pl.BlockSpec(memory_space=pltpu.VMEM),
        scratch_shapes=[
            pltpu.VMEM((2, K_SUB, 2, cm, subw), jnp.float32),
            pltpu.VMEM((2, K_SUB, 2, cm, subw), jnp.float32),
            pltpu.SemaphoreType.DMA((2, K_SUB, 2)),
            pltpu.SemaphoreType.DMA((2, K_SUB, 2)),
            pltpu.SemaphoreType.REGULAR((2, K_SUB)),
        ],
        compiler_params=pltpu.CompilerParams(collective_id=0),
    )(x, w_mat)
